# baseline (speedup 1.0000x reference)
import functools

import jax
import jax.numpy as jnp
from jax import lax
from jax.experimental import pallas as pl
from jax.experimental.pallas import tpu as pltpu
from jax.experimental.pallas import tpu_sc as plsc

_NW = 32
_CW = 2304
_NCHUNK = 434
_MAIN_END = _NCHUNK * _CW
_NCH_HI = -(-_NCHUNK // _NW)
_N_HI = _NCHUNK - _NW * (_NCH_HI - 1)
_B = 16384
_LBLK = 2048


def _cstart(w):
    return w * (_NCH_HI - 1) + jnp.minimum(w, _N_HI)


@functools.lru_cache(maxsize=None)
def _build():
    mesh = plsc.VectorSubcoreMesh(core_axis_name="c", subcore_axis_name="s")

    @functools.partial(
        pl.kernel,
        out_type=jax.ShapeDtypeStruct((_NW * 16,), jnp.float32),
        mesh=mesh,
        scratch_types=[
            pltpu.VMEM((16, _CW), jnp.float32),
            pltpu.VMEM((16, _CW), jnp.float32),
            pltpu.VMEM((_B + 16,), jnp.int32),
            pltpu.VMEM((_B + 16,), jnp.int32),
            pltpu.VMEM((_LBLK,), jnp.int32),
            pltpu.VMEM((16, 128), jnp.float32),
            pltpu.VMEM((8, 128), jnp.float32),
            pltpu.VMEM((16,), jnp.float32),
            pltpu.VMEM_SHARED((_B // 8, 128), jnp.float32),
            pltpu.SemaphoreType.DMA,
            pltpu.SemaphoreType.DMA,
        ],
        compiler_params=pltpu.CompilerParams(
            use_tc_tiling_on_sc=True, needs_layout_passes=False),
    )
    def sc_kernel(centerT, label, feat2048, side8, out,
                  buf0, buf1, mpack, cpack, labv, frows, side_v, accv,
                  feat_sh, sem_c, sem_f):
        wid = lax.axis_index("s") * 2 + lax.axis_index("c")
        sid = lax.axis_index("s")
        lane = lax.iota(jnp.int32, 16)

        c0 = _cstart(wid)
        lo = c0 * _CW
        hi = jnp.where(wid == _NW - 1, 1000000, _cstart(wid + 1) * _CW)

        def scalar(v):
            return lax.squeeze(lax.slice(v, (0,), (1,)), (0,))

        @pl.when(sid == 0)
        def _():
            pltpu.sync_copy(feat2048, feat_sh)
        pltpu.sync_copy(side8, side_v)

        start0 = jnp.minimum(c0 * _CW, _MAIN_END - _CW)
        pltpu.async_copy(centerT.at[:, pl.ds(start0, _CW)], buf0, sem_c)

        def pf_vreg(v, off, base):
            lv = labv[pl.ds(v * 16, 16)]
            pos = base + v * 16 + lane
            m = (lv >= lo) & (lv < hi)
            rel = lv - lo
            plsc.store_compressed(
                mpack.at[pl.ds(off, 16)], (rel << 14) | pos, mask=m)
            return off + scalar(plsc.all_reduce_population_count(m))

        def pf_step(s, off, base):
            off = pf_vreg(s * 2, off, base)
            return pf_vreg(s * 2 + 1, off, base)

        off = jnp.int32(0)
        for blk in range(_B // _LBLK):
            pltpu.sync_copy(label.at[pl.ds(blk * _LBLK, _LBLK)], labv)
            off = lax.fori_loop(
                0, _LBLK // 32,
                lambda s, o, b=blk * _LBLK: pf_step(s, o, b), off)
        mcnt = off
        mv = (mcnt + 15) >> 4

        plsc.subcore_barrier()

        def select_matches(rlo, rhi):
            def body(v, o):
                pk = mpack[pl.ds(v * 16, 16)]
                rel = pk >> 14
                valid = (v * 16 + lane) < mcnt
                m = (rel >= rlo) & (rel < rhi) & valid
                plsc.store_compressed(cpack.at[pl.ds(o, 16)], pk, mask=m)
                return o + scalar(plsc.all_reduce_population_count(m))
            return lax.fori_loop(0, mv, body, jnp.int32(0))

        def process_groups(ccnt, rcs, table_load, width, acc):
            def grp(gi, acc):
                pk = cpack[pl.ds(gi * 16, 16)]
                vmask = (gi * 16 + lane) < ccnt
                loc = pk >> 14
                loc = jnp.minimum(jnp.maximum(loc - rcs, 0), width - 1)
                pvec = pk & 16383
                pltpu.async_copy(
                    feat_sh.at[pvec >> 3], frows, sem_f).wait()
                fcol = (pvec & 7) << 4
                ga = jnp.zeros((16,), jnp.float32)
                for c in range(16):
                    cv = table_load(loc, c)
                    fv = plsc.load_gather(frows, [lane, fcol + c])
                    d = cv - fv
                    ga = ga + d * d
                return acc + jnp.where(vmask, ga, 0.0)
            return lax.fori_loop(0, (ccnt + 15) >> 4, grp, acc)

        acc = jnp.zeros((16,), jnp.float32)

        def chunk_iter(g, buf, nxt, acc):
            gl = jnp.minimum((c0 + g + 1) * _CW, _MAIN_END - _CW)
            pltpu.async_copy(centerT.at[:, pl.ds(gl, _CW)], nxt, sem_c)
            pltpu.make_async_copy(
                centerT.at[:, pl.ds(0, _CW)], buf, sem_c).wait()
            rcs = g * _CW
            ccnt = select_matches(
                rcs, jnp.minimum(rcs + _CW, _MAIN_END - lo))
            return process_groups(
                ccnt, rcs,
                lambda l, c: plsc.load_gather(
                    buf, [jnp.full((16,), c, jnp.int32), l]),
                _CW, acc)

        def outer(i, acc):
            acc = chunk_iter(2 * i, buf0, buf1, acc)
            return chunk_iter(2 * i + 1, buf1, buf0, acc)

        acc = lax.fori_loop(0, _NCH_HI // 2, outer, acc)
        pltpu.make_async_copy(
            centerT.at[:, pl.ds(0, _CW)], buf0, sem_c).wait()

        rts = _MAIN_END - lo
        tcnt = select_matches(rts, jnp.int32(2 ** 18))
        acc = process_groups(
            tcnt, rts,
            lambda l, c: plsc.load_gather(
                side_v, [((l << 4) + c) >> 7, ((l << 4) + c) & 127]),
            64, acc)

        accv[...] = acc
        pltpu.sync_copy(accv, out.at[pl.ds(wid * 16, 16)])

    return sc_kernel


def kernel(feature, label, center):
    B, D = feature.shape
    centerT = center.T
    feat2048 = feature.reshape(B * D // 128, 128)
    side8 = center[_MAIN_END:].reshape(8, 128)
    partials = _build()(centerT, label, feat2048, side8)
    return jnp.sqrt(jnp.sum(partials)) * (0.5 / B)


# speedup vs baseline: 1.0549x; 1.0549x over previous
import functools

import jax
import jax.numpy as jnp
from jax import lax
from jax.experimental import pallas as pl
from jax.experimental.pallas import tpu as pltpu
from jax.experimental.pallas import tpu_sc as plsc

_NW = 32
_CW = 2304
_NCHUNK = 434
_MAIN_END = _NCHUNK * _CW
_NCH_HI = -(-_NCHUNK // _NW)
_N_HI = _NCHUNK - _NW * (_NCH_HI - 1)
_B = 16384
_LBLK = 2048


def _cstart(w):
    return w * (_NCH_HI - 1) + jnp.minimum(w, _N_HI)


@functools.lru_cache(maxsize=None)
def _build():
    mesh = plsc.VectorSubcoreMesh(core_axis_name="c", subcore_axis_name="s")

    @functools.partial(
        pl.kernel,
        out_type=jax.ShapeDtypeStruct((_NW * 16,), jnp.float32),
        mesh=mesh,
        scratch_types=[
            pltpu.VMEM((16, _CW), jnp.float32),
            pltpu.VMEM((16, _CW), jnp.float32),
            pltpu.VMEM((_B + 16,), jnp.int32),
            pltpu.VMEM((_B + 16,), jnp.int32),
            pltpu.VMEM((_LBLK,), jnp.int32),
            pltpu.VMEM((_LBLK,), jnp.int32),
            pltpu.VMEM((16, 128), jnp.float32),
            pltpu.VMEM((8, 128), jnp.float32),
            pltpu.VMEM((16,), jnp.float32),
            pltpu.VMEM_SHARED((_B // 8, 128), jnp.float32),
            pltpu.SemaphoreType.DMA,
            pltpu.SemaphoreType.DMA,
        ],
        compiler_params=pltpu.CompilerParams(
            use_tc_tiling_on_sc=True, needs_layout_passes=False),
    )
    def sc_kernel(centerT, label, feat2048, side8, out,
                  buf0, buf1, mpack, cpack, labv, labv2, frows, side_v, accv,
                  feat_sh, sem_c, sem_f):
        wid = lax.axis_index("s") * 2 + lax.axis_index("c")
        sid = lax.axis_index("s")
        lane = lax.iota(jnp.int32, 16)

        c0 = _cstart(wid)
        lo = c0 * _CW
        hi = jnp.where(wid == _NW - 1, 1000000, _cstart(wid + 1) * _CW)

        def scalar(v):
            return lax.squeeze(lax.slice(v, (0,), (1,)), (0,))

        @pl.when(sid == 0)
        def _():
            pltpu.sync_copy(feat2048, feat_sh)
        pltpu.sync_copy(side8, side_v)

        start0 = jnp.minimum(c0 * _CW, _MAIN_END - _CW)
        pltpu.async_copy(centerT.at[:, pl.ds(start0, _CW)], buf0, sem_c)

        def pf_vreg(v, off, base, lref):
            lv = lref[pl.ds(v * 16, 16)]
            pos = base + v * 16 + lane
            m = (lv >= lo) & (lv < hi)
            rel = lv - lo
            plsc.store_compressed(
                mpack.at[pl.ds(off, 16)], (rel << 14) | pos, mask=m)
            return off + scalar(plsc.all_reduce_population_count(m))

        def pf_step(s, off, base, lref):
            off = pf_vreg(s * 2, off, base, lref)
            return pf_vreg(s * 2 + 1, off, base, lref)

        lbufs = (labv, labv2)
        nblk = _B // _LBLK
        copies = [pltpu.async_copy(
            label.at[pl.ds(0, _LBLK)], labv, sem_f)]
        off = jnp.int32(0)
        for blk in range(nblk):
            if blk + 1 < nblk:
                copies.append(pltpu.async_copy(
                    label.at[pl.ds((blk + 1) * _LBLK, _LBLK)],
                    lbufs[(blk + 1) % 2], sem_f))
            copies[blk].wait()
            off = lax.fori_loop(
                0, _LBLK // 32,
                lambda s, o, b=blk * _LBLK, r=lbufs[blk % 2]: pf_step(
                    s, o, b, r), off)
        mcnt = off
        mv = (mcnt + 15) >> 4

        plsc.subcore_barrier()

        def select_matches(rlo, rhi):
            def body(v, o):
                pk = mpack[pl.ds(v * 16, 16)]
                rel = pk >> 14
                valid = (v * 16 + lane) < mcnt
                m = (rel >= rlo) & (rel < rhi) & valid
                plsc.store_compressed(cpack.at[pl.ds(o, 16)], pk, mask=m)
                return o + scalar(plsc.all_reduce_population_count(m))
            return lax.fori_loop(0, mv, body, jnp.int32(0))

        def process_groups(ccnt, rcs, table_load, width, acc):
            def grp(gi, acc):
                pk = cpack[pl.ds(gi * 16, 16)]
                vmask = (gi * 16 + lane) < ccnt
                loc = pk >> 14
                loc = jnp.minimum(jnp.maximum(loc - rcs, 0), width - 1)
                pvec = pk & 16383
                pltpu.async_copy(
                    feat_sh.at[pvec >> 3], frows, sem_f).wait()
                fcol = (pvec & 7) << 4
                ga = jnp.zeros((16,), jnp.float32)
                for c in range(16):
                    cv = table_load(loc, c)
                    fv = plsc.load_gather(frows, [lane, fcol + c])
                    d = cv - fv
                    ga = ga + d * d
                return acc + jnp.where(vmask, ga, 0.0)
            return lax.fori_loop(0, (ccnt + 15) >> 4, grp, acc)

        acc = jnp.zeros((16,), jnp.float32)

        def chunk_iter(g, buf, nxt, acc):
            gl = jnp.minimum((c0 + g + 1) * _CW, _MAIN_END - _CW)
            pltpu.async_copy(centerT.at[:, pl.ds(gl, _CW)], nxt, sem_c)
            pltpu.make_async_copy(
                centerT.at[:, pl.ds(0, _CW)], buf, sem_c).wait()
            rcs = g * _CW
            ccnt = select_matches(
                rcs, jnp.minimum(rcs + _CW, _MAIN_END - lo))
            return process_groups(
                ccnt, rcs,
                lambda l, c: plsc.load_gather(
                    buf, [jnp.full((16,), c, jnp.int32), l]),
                _CW, acc)

        def outer(i, acc):
            acc = chunk_iter(2 * i, buf0, buf1, acc)
            return chunk_iter(2 * i + 1, buf1, buf0, acc)

        acc = lax.fori_loop(0, _NCH_HI // 2, outer, acc)
        pltpu.make_async_copy(
            centerT.at[:, pl.ds(0, _CW)], buf0, sem_c).wait()

        rts = _MAIN_END - lo
        tcnt = select_matches(rts, jnp.int32(2 ** 18))
        acc = process_groups(
            tcnt, rts,
            lambda l, c: plsc.load_gather(
                side_v, [((l << 4) + c) >> 7, ((l << 4) + c) & 127]),
            64, acc)

        accv[...] = acc
        pltpu.sync_copy(accv, out.at[pl.ds(wid * 16, 16)])

    return sc_kernel


def kernel(feature, label, center):
    B, D = feature.shape
    centerT = center.T
    feat2048 = feature.reshape(B * D // 128, 128)
    side8 = center[_MAIN_END:].reshape(8, 128)
    partials = _build()(centerT, label, feat2048, side8)
    return jnp.sqrt(jnp.sum(partials)) * (0.5 / B)


# speedup vs baseline: 1.0707x; 1.0150x over previous
import functools

import jax
import jax.numpy as jnp
from jax import lax
from jax.experimental import pallas as pl
from jax.experimental.pallas import tpu as pltpu
from jax.experimental.pallas import tpu_sc as plsc

_NW = 32
_CW = 1536
_NCHUNK = 651
_MAIN_END = _NCHUNK * _CW
_NCH_HI = -(-_NCHUNK // _NW)
_N_HI = _NCHUNK - _NW * (_NCH_HI - 1)
_B = 16384
_LBLK = 2048


def _cstart(w):
    return w * (_NCH_HI - 1) + jnp.minimum(w, _N_HI)


@functools.lru_cache(maxsize=None)
def _build():
    mesh = plsc.VectorSubcoreMesh(core_axis_name="c", subcore_axis_name="s")

    @functools.partial(
        pl.kernel,
        out_type=jax.ShapeDtypeStruct((_NW * 16,), jnp.float32),
        mesh=mesh,
        scratch_types=[
            pltpu.VMEM((16, _CW), jnp.float32),
            pltpu.VMEM((16, _CW), jnp.float32),
            pltpu.VMEM((16, _CW), jnp.float32),
            pltpu.VMEM((_B + 16,), jnp.int32),
            pltpu.VMEM((_B + 16,), jnp.int32),
            pltpu.VMEM((_LBLK,), jnp.int32),
            pltpu.VMEM((_LBLK,), jnp.int32),
            pltpu.VMEM((16, 128), jnp.float32),
            pltpu.VMEM((8, 128), jnp.float32),
            pltpu.VMEM((16,), jnp.float32),
            pltpu.VMEM_SHARED((_B // 8, 128), jnp.float32),
            pltpu.SemaphoreType.DMA,
            pltpu.SemaphoreType.DMA,
        ],
        compiler_params=pltpu.CompilerParams(
            use_tc_tiling_on_sc=True, needs_layout_passes=False),
    )
    def sc_kernel(centerT, label, feat2048, side8, out,
                  buf0, buf1, buf2, mpack, cpack, labv, labv2, frows, side_v, accv,
                  feat_sh, sem_c, sem_f):
        wid = lax.axis_index("s") * 2 + lax.axis_index("c")
        sid = lax.axis_index("s")
        lane = lax.iota(jnp.int32, 16)

        c0 = _cstart(wid)
        lo = c0 * _CW
        hi = jnp.where(wid == _NW - 1, 1000000, _cstart(wid + 1) * _CW)

        def scalar(v):
            return lax.squeeze(lax.slice(v, (0,), (1,)), (0,))

        @pl.when(sid == 0)
        def _():
            pltpu.sync_copy(feat2048, feat_sh)
        pltpu.sync_copy(side8, side_v)

        def cslice(g):
            gl = jnp.minimum((c0 + g) * _CW, _MAIN_END - _CW)
            return centerT.at[:, pl.ds(gl, _CW)]

        pltpu.async_copy(cslice(0), buf0, sem_c)
        pltpu.async_copy(cslice(1), buf1, sem_c)

        def pf_vreg(v, off, base, lref):
            lv = lref[pl.ds(v * 16, 16)]
            pos = base + v * 16 + lane
            m = (lv >= lo) & (lv < hi)
            rel = lv - lo
            plsc.store_compressed(
                mpack.at[pl.ds(off, 16)], (rel << 14) | pos, mask=m)
            return off + scalar(plsc.all_reduce_population_count(m))

        def pf_step(s, off, base, lref):
            off = pf_vreg(s * 2, off, base, lref)
            return pf_vreg(s * 2 + 1, off, base, lref)

        lbufs = (labv, labv2)
        nblk = _B // _LBLK
        copies = [pltpu.async_copy(
            label.at[pl.ds(0, _LBLK)], labv, sem_f)]
        off = jnp.int32(0)
        for blk in range(nblk):
            if blk + 1 < nblk:
                copies.append(pltpu.async_copy(
                    label.at[pl.ds((blk + 1) * _LBLK, _LBLK)],
                    lbufs[(blk + 1) % 2], sem_f))
            copies[blk].wait()
            off = lax.fori_loop(
                0, _LBLK // 32,
                lambda s, o, b=blk * _LBLK, r=lbufs[blk % 2]: pf_step(
                    s, o, b, r), off)
        mcnt = off
        mv = (mcnt + 15) >> 4

        plsc.subcore_barrier()

        def select_matches(rlo, rhi):
            def body(v, o):
                pk = mpack[pl.ds(v * 16, 16)]
                rel = pk >> 14
                valid = (v * 16 + lane) < mcnt
                m = (rel >= rlo) & (rel < rhi) & valid
                plsc.store_compressed(cpack.at[pl.ds(o, 16)], pk, mask=m)
                return o + scalar(plsc.all_reduce_population_count(m))
            return lax.fori_loop(0, mv, body, jnp.int32(0))

        def process_groups(ccnt, rcs, table_load, width, acc):
            def grp(gi, acc):
                pk = cpack[pl.ds(gi * 16, 16)]
                vmask = (gi * 16 + lane) < ccnt
                loc = pk >> 14
                loc = jnp.minimum(jnp.maximum(loc - rcs, 0), width - 1)
                pvec = pk & 16383
                pltpu.async_copy(
                    feat_sh.at[pvec >> 3], frows, sem_f).wait()
                fcol = (pvec & 7) << 4
                ga = jnp.zeros((16,), jnp.float32)
                for c in range(16):
                    cv = table_load(loc, c)
                    fv = plsc.load_gather(frows, [lane, fcol + c])
                    d = cv - fv
                    ga = ga + d * d
                return acc + jnp.where(vmask, ga, 0.0)
            return lax.fori_loop(0, (ccnt + 15) >> 4, grp, acc)

        acc = jnp.zeros((16,), jnp.float32)

        def chunk_iter(g, buf, nxt, acc):
            pltpu.async_copy(cslice(g + 2), nxt, sem_c)
            pltpu.make_async_copy(
                centerT.at[:, pl.ds(0, _CW)], buf, sem_c).wait()
            rcs = g * _CW
            ccnt = select_matches(
                rcs, jnp.minimum(rcs + _CW, _MAIN_END - lo))
            return process_groups(
                ccnt, rcs,
                lambda l, c: plsc.load_gather(
                    buf, [jnp.full((16,), c, jnp.int32), l]),
                _CW, acc)

        def outer(i, acc):
            acc = chunk_iter(3 * i, buf0, buf2, acc)
            acc = chunk_iter(3 * i + 1, buf1, buf0, acc)
            return chunk_iter(3 * i + 2, buf2, buf1, acc)

        acc = lax.fori_loop(0, _NCH_HI // 3, outer, acc)
        for _ in range(2):
            pltpu.make_async_copy(
                centerT.at[:, pl.ds(0, _CW)], buf0, sem_c).wait()

        rts = _MAIN_END - lo
        tcnt = select_matches(rts, jnp.int32(2 ** 18))
        acc = process_groups(
            tcnt, rts,
            lambda l, c: plsc.load_gather(
                side_v, [((l << 4) + c) >> 7, ((l << 4) + c) & 127]),
            64, acc)

        accv[...] = acc
        pltpu.sync_copy(accv, out.at[pl.ds(wid * 16, 16)])

    return sc_kernel


def kernel(feature, label, center):
    B, D = feature.shape
    centerT = center.T
    feat2048 = feature.reshape(B * D // 128, 128)
    side8 = center[_MAIN_END:].reshape(8, 128)
    partials = _build()(centerT, label, feat2048, side8)
    return jnp.sqrt(jnp.sum(partials)) * (0.5 / B)
